# baseline (speedup 1.0000x reference)
import jax
import jax.numpy as jnp
from jax import lax
from jax.experimental import pallas as pl
from jax.experimental.pallas import tpu as pltpu

HIDDEN = 2048
NUM_EXPERTS = 16
TOP_K = 2
LOAD_BALANCE_ALPHA = 0.01
Z_LOSS_ALPHA = 0.0001
EXPERT_CAPACITY_LIMIT = 1.2

TOKEN_TILE = 1024
NBUF = 4


def _gate_kernel(x_ref, w_ref, ts_ref, ti_ref, loss_ref,
                 xbuf_ref, scores_ref, acc_ref, sems):
    n = scores_ref.shape[1]
    n_tiles = n // TOKEN_TILE
    n_tokens = n

    def copy_in(k):
        return pltpu.make_async_copy(
            x_ref.at[pl.ds(k * TOKEN_TILE, TOKEN_TILE), :],
            xbuf_ref.at[k % NBUF],
            sems.at[k % NBUF],
        )

    for k in range(min(NBUF, n_tiles)):
        copy_in(k).start()

    w = w_ref[...]
    acc_load = jnp.zeros((NUM_EXPERTS, 1), jnp.float32)
    acc_z = jnp.zeros((1, 1), jnp.float32)

    for k in range(n_tiles):
        copy_in(k).wait()
        logits = jnp.dot(xbuf_ref[k % NBUF], w,
                         precision=lax.Precision.DEFAULT,
                         preferred_element_type=jnp.float32)
        if k + NBUF < n_tiles:
            copy_in(k + NBUF).start()
        lt = logits.T
        m = jnp.max(lt, axis=0, keepdims=True)
        ex = jnp.exp(lt - m)
        se = jnp.sum(ex, axis=0, keepdims=True)
        st = ex / se
        scores_ref[:, pl.ds(k * TOKEN_TILE, TOKEN_TILE)] = st
        acc_load = acc_load + jnp.sum(st, axis=1, keepdims=True)
        lse = m + jnp.log(se)
        acc_z = acc_z + jnp.sum(lse * lse, axis=1, keepdims=True)

    acc_ref[:, 0:1] = acc_load
    acc_ref[0:1, 1:2] = acc_z

    scores = scores_ref[...]
    load = acc_ref[:, 0:1]
    capacity = EXPERT_CAPACITY_LIMIT * n_tokens / NUM_EXPERTS
    scale = jnp.minimum(1.0, capacity / (load + 1e-8))
    scaled = scores * scale
    colsum = jnp.sum(scaled, axis=0, keepdims=True)
    r = scaled / (colsum + 1e-8)

    iota = lax.broadcasted_iota(jnp.int32, r.shape, 0)
    m1 = jnp.max(r, axis=0, keepdims=True)
    i1 = jnp.min(jnp.where(r == m1, iota, NUM_EXPERTS),
                 axis=0, keepdims=True)
    masked = jnp.where(iota == i1, -jnp.inf, r)
    m2 = jnp.max(masked, axis=0, keepdims=True)
    i2 = jnp.min(jnp.where(masked == m2, iota, NUM_EXPERTS),
                 axis=0, keepdims=True)
    e = jnp.exp(m2 - m1)
    inv = 1.0 / (1.0 + e)
    ts_ref[...] = jnp.concatenate([inv, e * inv], axis=0)
    ti_ref[...] = jnp.concatenate([i1, i2], axis=0)

    mean_prob = jnp.sum(r, axis=1, keepdims=True) / n_tokens
    frac = jnp.sum(jnp.where(iota == i1, 1.0, 0.0),
                   axis=1, keepdims=True) / n_tokens
    lb = (LOAD_BALANCE_ALPHA * NUM_EXPERTS
          * jnp.sum(frac * mean_prob, axis=0, keepdims=True))
    z = Z_LOSS_ALPHA * acc_ref[0:1, 1:2] / n_tokens
    loss_ref[...] = lb + z


def kernel(x, W):
    b, s, h = x.shape
    n = b * s
    x_flat = x.reshape(n, h)
    wt = W.T

    ts_t, ti_t, loss = pl.pallas_call(
        _gate_kernel,
        in_specs=[
            pl.BlockSpec(memory_space=pltpu.MemorySpace.HBM),
            pl.BlockSpec(memory_space=pltpu.MemorySpace.VMEM),
        ],
        out_specs=[
            pl.BlockSpec(memory_space=pltpu.MemorySpace.VMEM),
            pl.BlockSpec(memory_space=pltpu.MemorySpace.VMEM),
            pl.BlockSpec(memory_space=pltpu.MemorySpace.VMEM),
        ],
        out_shape=[
            jax.ShapeDtypeStruct((TOP_K, n), jnp.float32),
            jax.ShapeDtypeStruct((TOP_K, n), jnp.int32),
            jax.ShapeDtypeStruct((1, 1), jnp.float32),
        ],
        scratch_shapes=[
            pltpu.VMEM((NBUF, TOKEN_TILE, HIDDEN), jnp.float32),
            pltpu.VMEM((NUM_EXPERTS, n), jnp.float32),
            pltpu.VMEM((NUM_EXPERTS, 128), jnp.float32),
            pltpu.SemaphoreType.DMA((NBUF,)),
        ],
    )(x_flat, wt)

    return ts_t.T, ti_t.T, loss.reshape(())


# speedup vs baseline: 1.0846x; 1.0846x over previous
import jax
import jax.numpy as jnp
from jax import lax
from jax.experimental import pallas as pl
from jax.experimental.pallas import tpu as pltpu

HIDDEN = 2048
NUM_EXPERTS = 16
TOP_K = 2
LOAD_BALANCE_ALPHA = 0.01
Z_LOSS_ALPHA = 0.0001
EXPERT_CAPACITY_LIMIT = 1.2

TOKEN_TILE = 1024


def _gate_kernel(x_ref, w_ref, ts_ref, ti_ref, loss_ref, scores_ref, acc_ref):
    i = pl.program_id(0)
    n_tiles = pl.num_programs(0) - 1
    n_tokens = n_tiles * TOKEN_TILE

    @pl.when(i == 0)
    def _init():
        acc_ref[...] = jnp.zeros_like(acc_ref)

    @pl.when(i < n_tiles)
    def _matmul_step():
        logits = jnp.dot(x_ref[...], w_ref[...],
                         precision=lax.Precision.DEFAULT,
                         preferred_element_type=jnp.float32)
        lt = logits.T
        m = jnp.max(lt, axis=0, keepdims=True)
        ex = jnp.exp(lt - m)
        se = jnp.sum(ex, axis=0, keepdims=True)
        st = ex / se
        scores_ref[:, pl.ds(i * TOKEN_TILE, TOKEN_TILE)] = st
        acc_ref[:, 0:1] += jnp.sum(st, axis=1, keepdims=True)
        lse = m + jnp.log(se)
        acc_ref[0:1, 1:2] += jnp.sum(lse * lse, axis=1, keepdims=True)

    @pl.when(i == n_tiles)
    def _finalize():
        scores = scores_ref[...]
        load = acc_ref[:, 0:1]
        capacity = EXPERT_CAPACITY_LIMIT * n_tokens / NUM_EXPERTS
        scale = jnp.minimum(1.0, capacity / (load + 1e-8))
        scaled = scores * scale
        colsum = jnp.sum(scaled, axis=0, keepdims=True)
        r = scaled / (colsum + 1e-8)

        iota = lax.broadcasted_iota(jnp.int32, r.shape, 0)
        m1 = jnp.max(r, axis=0, keepdims=True)
        i1 = jnp.min(jnp.where(r == m1, iota, NUM_EXPERTS),
                     axis=0, keepdims=True)
        masked = jnp.where(iota == i1, -jnp.inf, r)
        m2 = jnp.max(masked, axis=0, keepdims=True)
        i2 = jnp.min(jnp.where(masked == m2, iota, NUM_EXPERTS),
                     axis=0, keepdims=True)
        e = jnp.exp(m2 - m1)
        inv = 1.0 / (1.0 + e)
        ts_ref[...] = jnp.concatenate([inv, e * inv], axis=0)
        ti_ref[...] = jnp.concatenate([i1, i2], axis=0)

        mean_prob = jnp.sum(r, axis=1, keepdims=True) / n_tokens
        frac = jnp.sum(jnp.where(iota == i1, 1.0, 0.0),
                       axis=1, keepdims=True) / n_tokens
        lb = (LOAD_BALANCE_ALPHA * NUM_EXPERTS
              * jnp.sum(frac * mean_prob, axis=0, keepdims=True))
        z = Z_LOSS_ALPHA * acc_ref[0:1, 1:2] / n_tokens
        loss_ref[...] = lb + z


def kernel(x, W):
    b, s, h = x.shape
    n = b * s
    x_flat = x.reshape(n, h)
    wt = W.T
    n_tiles = n // TOKEN_TILE

    ts_t, ti_t, loss = pl.pallas_call(
        _gate_kernel,
        grid=(n_tiles + 1,),
        in_specs=[
            pl.BlockSpec((TOKEN_TILE, HIDDEN),
                         lambda i: (jnp.minimum(i, n_tiles - 1), 0)),
            pl.BlockSpec((HIDDEN, NUM_EXPERTS), lambda i: (0, 0)),
        ],
        out_specs=[
            pl.BlockSpec((TOP_K, n), lambda i: (0, 0)),
            pl.BlockSpec((TOP_K, n), lambda i: (0, 0)),
            pl.BlockSpec((1, 1), lambda i: (0, 0)),
        ],
        out_shape=[
            jax.ShapeDtypeStruct((TOP_K, n), jnp.float32),
            jax.ShapeDtypeStruct((TOP_K, n), jnp.int32),
            jax.ShapeDtypeStruct((1, 1), jnp.float32),
        ],
        scratch_shapes=[
            pltpu.VMEM((NUM_EXPERTS, n), jnp.float32),
            pltpu.VMEM((NUM_EXPERTS, 128), jnp.float32),
        ],
    )(x_flat, wt)

    return ts_t.T, ti_t.T, loss.reshape(())
